# Initial kernel scaffold; baseline (speedup 1.0000x reference)
import jax
import jax.numpy as jnp
from jax.experimental import pallas as pl


def kernel(feat, row_ids, num_rows):
    raise NotImplementedError("write your pallas kernel here")


# speedup vs baseline: 1.3645x; 1.3645x over previous
import functools

import jax
import jax.numpy as jnp
from jax import lax
from jax.experimental import pallas as pl
from jax.experimental.pallas import tpu as pltpu
from jax.experimental.pallas import tpu_sc as plsc

N = 320000
D = 128
R = 10000
NC = 2
NS = 16
NW = NC * NS
CHUNK = N // NW
BLKA = 200
NBLKA = CHUNK // BLKA
BLKB = 80
NBLKB = CHUNK // BLKB
MROWS = R // 16

_mesh = plsc.VectorSubcoreMesh(
    core_axis_name="c", subcore_axis_name="s", num_cores=NC, num_subcores=NS)


def _wid():
    return lax.axis_index("s") * NC + lax.axis_index("c")


def _vecs(ref, *idx):
    return [ref[(*idx, pl.ds(16 * k, 16))] for k in range(8)]


def _store_row(ref, vals, *idx):
    for k in range(8):
        ref[(*idx, pl.ds(16 * k, 16))] = vals[k]


@functools.partial(
    pl.kernel,
    out_type=(
        jax.ShapeDtypeStruct((R * D,), jnp.float32),
        jax.ShapeDtypeStruct((2 * NW * D,), jnp.float32),
        jax.ShapeDtypeStruct((2 * NW * 16,), jnp.int32),
    ),
    mesh=_mesh,
    scratch_types=[
        pltpu.VMEM((CHUNK + 16,), jnp.int32),
        pltpu.VMEM((BLKA, D), jnp.float32),
        pltpu.VMEM((D,), jnp.float32),
        pltpu.VMEM((D,), jnp.float32),
        pltpu.VMEM((32,), jnp.int32),
    ],
)
def _phase_a(feat_hbm, ids_hbm, table_hbm, sidev_hbm, sidei_hbm,
             ids_v, fbuf, stage, first_acc, idstage):
    w = _wid()
    base = w * CHUNK
    pltpu.sync_copy(ids_hbm.at[pl.ds(base, CHUNK)], ids_v.at[pl.ds(0, CHUNK)])

    neg = jnp.full((16,), -jnp.inf, jnp.float32)
    carry = (neg,) * 8 + (ids_v[pl.ds(0, 16)][0], jnp.int32(0))

    for blk in range(NBLKA):
        pltpu.sync_copy(feat_hbm.at[pl.ds(base + blk * BLKA, BLKA)], fbuf)

        def row_body(j, carry, blk=blk):
            accs, cur_id, nflush = carry[:8], carry[8], carry[9]
            sid = ids_v[pl.ds(blk * BLKA + j, 16)][0]
            changed = sid != cur_id

            @pl.when(changed & (nflush == 0))
            def _():
                _store_row(first_acc, accs)

            @pl.when(changed & (nflush > 0))
            def _():
                _store_row(stage, accs)
                pltpu.sync_copy(stage, table_hbm.at[pl.ds(cur_id * D, D)])

            f = _vecs(fbuf, j)
            pen = jnp.broadcast_to(
                jnp.where(changed, jnp.float32(-jnp.inf), jnp.float32(0.0)),
                (16,))
            new_accs = tuple(jnp.maximum(accs[k] + pen, f[k]) for k in range(8))
            new_cur = jnp.where(changed, sid, cur_id)
            return new_accs + (new_cur, nflush + changed.astype(jnp.int32))

        carry = lax.fori_loop(0, BLKA, row_body, carry)

    accs, nflush = carry[:8], carry[9]
    _store_row(stage, accs)

    @pl.when(nflush == 0)
    def _():
        _store_row(first_acc, accs)

    pltpu.sync_copy(first_acc, sidev_hbm.at[pl.ds(2 * w * D, D)])
    pltpu.sync_copy(stage, sidev_hbm.at[pl.ds((2 * w + 1) * D, D)])
    idstage[pl.ds(0, 16)] = jnp.broadcast_to(ids_v[pl.ds(0, 16)][0], (16,))
    idstage[pl.ds(16, 16)] = jnp.broadcast_to(ids_v[pl.ds(CHUNK - 16, 16)][15], (16,))
    pltpu.sync_copy(idstage, sidei_hbm.at[pl.ds(2 * w * 16, 32)])


@functools.partial(
    pl.kernel,
    out_type=jax.ShapeDtypeStruct((R * D,), jnp.float32),
    mesh=_mesh,
    scratch_types=[
        pltpu.VMEM((2 * NW * D,), jnp.float32),
        pltpu.VMEM((2 * NW * 16,), jnp.int32),
        pltpu.VMEM((MROWS * D,), jnp.float32),
    ],
)
def _phase_m(tin_hbm, sidev_hbm, sidei_hbm, tout_hbm, svv, siv, tbuf):
    w = _wid()

    @pl.when(w < 16)
    def _():
        lo = w * MROWS

        def row(e):
            return [svv[pl.ds(e * D + 16 * k, 16)] for k in range(8)]

        def set_row(e, vals):
            for k in range(8):
                svv[pl.ds(e * D + 16 * k, 16)] = vals[k]

        pltpu.sync_copy(tin_hbm.at[pl.ds(lo * D, MROWS * D)], tbuf)
        pltpu.sync_copy(sidev_hbm, svv)
        pltpu.sync_copy(sidei_hbm, siv)
        sids = [siv[pl.ds(e * 16, 16)][0] for e in range(2 * NW)]
        for e in range(1, 2 * NW):
            same = sids[e] == sids[e - 1]
            pen = jnp.broadcast_to(
                jnp.where(same, jnp.float32(0.0), jnp.float32(-jnp.inf)), (16,))
            prev, cur = row(e - 1), row(e)
            set_row(e, [jnp.maximum(cur[k], prev[k] + pen) for k in range(8)])
        for e in range(2 * NW - 2, -1, -1):
            same = sids[e] == sids[e + 1]
            pen_c = jnp.broadcast_to(
                jnp.where(same, jnp.float32(-jnp.inf), jnp.float32(0.0)), (16,))
            pen_n = jnp.broadcast_to(
                jnp.where(same, jnp.float32(0.0), jnp.float32(-jnp.inf)), (16,))
            nxt, cur = row(e + 1), row(e)
            set_row(e, [jnp.maximum(cur[k] + pen_c, nxt[k] + pen_n)
                        for k in range(8)])
        for e in range(2 * NW):
            sid = sids[e]

            @pl.when((sid >= lo) & (sid < lo + MROWS))
            def _(e=e, sid=sid):
                off = (sid - lo) * D
                for k in range(8):
                    tbuf[pl.ds(off + 16 * k, 16)] = svv[pl.ds(e * D + 16 * k, 16)]

        pltpu.sync_copy(tbuf, tout_hbm.at[pl.ds(lo * D, MROWS * D)])


@functools.partial(
    pl.kernel,
    out_type=jax.ShapeDtypeStruct((N, 2, D), jnp.float32),
    mesh=_mesh,
    scratch_types=[
        pltpu.VMEM((CHUNK,), jnp.int32),
        pltpu.VMEM((BLKB, D), jnp.float32),
        pltpu.VMEM((BLKB, D), jnp.float32),
        pltpu.VMEM((BLKB, 2, D), jnp.float32),
        pltpu.SemaphoreType.DMA,
    ],
)
def _phase_b(feat_hbm, ids_hbm, table_hbm, out_hbm,
             ids_v, fbuf, gbuf, obuf, sem):
    w = _wid()
    base = w * CHUNK
    pltpu.sync_copy(ids_hbm.at[pl.ds(base, CHUNK)], ids_v)

    def blk_body(b, carry):
        r0 = base + b * BLKB
        pltpu.async_copy(
            table_hbm.at[ids_v.at[pl.ds(b * BLKB, BLKB)]], gbuf, sem).wait()
        pltpu.sync_copy(feat_hbm.at[pl.ds(r0, BLKB)], fbuf)

        def row_body(j, carry):
            f = _vecs(fbuf, j)
            g = _vecs(gbuf, j)
            _store_row(obuf, [g[k] - f[k] for k in range(8)], j, 0)
            _store_row(obuf, f, j, 1)
            return carry

        lax.fori_loop(0, BLKB, row_body, carry)
        pltpu.sync_copy(obuf, out_hbm.at[pl.ds(r0, BLKB)])
        return carry

    lax.fori_loop(0, NBLKB, blk_body, jnp.int32(0))


def kernel(feat, row_ids, num_rows):
    table, sidev, sidei = _phase_a(feat, row_ids)
    table = _phase_m(table, sidev, sidei)
    return _phase_b(feat, row_ids, table.reshape(R, D))


# speedup vs baseline: 1.8635x; 1.3657x over previous
import functools

import jax
import jax.numpy as jnp
from jax import lax
from jax.experimental import pallas as pl
from jax.experimental.pallas import tpu as pltpu
from jax.experimental.pallas import tpu_sc as plsc

N = 320000
D = 128
R = 10000
NC = 2
NS = 16
NW = NC * NS
CHUNK = N // NW
BLKA = 200
NBLKA = CHUNK // BLKA
BLKB = 80
NBLKB = CHUNK // BLKB
MROWS = R // 16

_mesh = plsc.VectorSubcoreMesh(
    core_axis_name="c", subcore_axis_name="s", num_cores=NC, num_subcores=NS)


def _wid():
    return lax.axis_index("s") * NC + lax.axis_index("c")


def _vecs(ref, *idx):
    return [ref[(*idx, pl.ds(16 * k, 16))] for k in range(8)]


def _store_row(ref, vals, *idx):
    for k in range(8):
        ref[(*idx, pl.ds(16 * k, 16))] = vals[k]


NSTAGE = 4

@functools.partial(
    pl.kernel,
    out_type=(
        jax.ShapeDtypeStruct((R * D,), jnp.float32),
        jax.ShapeDtypeStruct((2 * NW * D,), jnp.float32),
        jax.ShapeDtypeStruct((2 * NW * 16,), jnp.int32),
    ),
    mesh=_mesh,
    scratch_types=[
        pltpu.VMEM((CHUNK + 16,), jnp.int32),
        pltpu.VMEM((2 * BLKA, D), jnp.float32),
        pltpu.VMEM((NSTAGE * D,), jnp.float32),
        pltpu.VMEM((D,), jnp.float32),
        pltpu.VMEM((32,), jnp.int32),
        pltpu.SemaphoreType.DMA,
        pltpu.SemaphoreType.DMA,
    ],
)
def _phase_a(feat_hbm, ids_hbm, table_hbm, sidev_hbm, sidei_hbm,
             ids_v, fbuf, stage, first_acc, idstage, sem_f, sem_t):
    w = _wid()
    base = w * CHUNK
    pltpu.sync_copy(ids_hbm.at[pl.ds(base, CHUNK)], ids_v.at[pl.ds(0, CHUNK)])

    neg = jnp.full((16,), -jnp.inf, jnp.float32)
    carry = (neg,) * 8 + (ids_v[pl.ds(0, 16)][0], jnp.int32(0))

    pltpu.async_copy(
        feat_hbm.at[pl.ds(base, BLKA)], fbuf.at[pl.ds(0, BLKA)], sem_f)
    for blk in range(NBLKA):
        slot = blk % 2
        pltpu.make_async_copy(
            feat_hbm.at[pl.ds(base, BLKA)], fbuf.at[pl.ds(slot * BLKA, BLKA)],
            sem_f).wait()
        if blk + 1 < NBLKA:
            pltpu.async_copy(
                feat_hbm.at[pl.ds(base + (blk + 1) * BLKA, BLKA)],
                fbuf.at[pl.ds((1 - slot) * BLKA, BLKA)], sem_f)

        def row_body(j, carry, blk=blk, slot=slot):
            accs, cur_id, nflush = carry[:8], carry[8], carry[9]
            sid = ids_v[pl.ds(blk * BLKA + j, 16)][0]
            changed = sid != cur_id

            @pl.when(changed & (nflush == 0))
            def _():
                _store_row(first_acc, accs)

            @pl.when(changed & (nflush > 0))
            def _():
                islot = (nflush - 1) % NSTAGE

                @pl.when(nflush > NSTAGE)
                def _():
                    pltpu.make_async_copy(
                        stage.at[pl.ds(0, D)], table_hbm.at[pl.ds(0, D)],
                        sem_t).wait()

                soff = islot * D
                for k in range(8):
                    stage[pl.ds(soff + 16 * k, 16)] = accs[k]
                pltpu.async_copy(
                    stage.at[pl.ds(soff, D)],
                    table_hbm.at[pl.ds(cur_id * D, D)], sem_t)

            f = _vecs(fbuf, slot * BLKA + j)
            pen = jnp.broadcast_to(
                jnp.where(changed, jnp.float32(-jnp.inf), jnp.float32(0.0)),
                (16,))
            new_accs = tuple(jnp.maximum(accs[k] + pen, f[k]) for k in range(8))
            new_cur = jnp.where(changed, sid, cur_id)
            return new_accs + (new_cur, nflush + changed.astype(jnp.int32))

        carry = lax.fori_loop(0, BLKA, row_body, carry)

    accs, nflush = carry[:8], carry[9]

    @pl.when(nflush == 0)
    def _():
        _store_row(first_acc, accs)

    ndrain = jnp.minimum(jnp.maximum(nflush - 1, 0), NSTAGE)

    def drain(i, c):
        pltpu.make_async_copy(
            stage.at[pl.ds(0, D)], table_hbm.at[pl.ds(0, D)], sem_t).wait()
        return c

    lax.fori_loop(0, ndrain, drain, jnp.int32(0))
    _store_row(stage, accs)

    pltpu.sync_copy(first_acc, sidev_hbm.at[pl.ds(2 * w * D, D)])
    pltpu.sync_copy(stage.at[pl.ds(0, D)],
                    sidev_hbm.at[pl.ds((2 * w + 1) * D, D)])
    idstage[pl.ds(0, 16)] = jnp.broadcast_to(ids_v[pl.ds(0, 16)][0], (16,))
    idstage[pl.ds(16, 16)] = jnp.broadcast_to(ids_v[pl.ds(CHUNK - 16, 16)][15], (16,))
    pltpu.sync_copy(idstage, sidei_hbm.at[pl.ds(2 * w * 16, 32)])


@functools.partial(
    pl.kernel,
    out_type=jax.ShapeDtypeStruct((R * D,), jnp.float32),
    mesh=_mesh,
    scratch_types=[
        pltpu.VMEM((2 * NW * D,), jnp.float32),
        pltpu.VMEM((2 * NW * 16,), jnp.int32),
        pltpu.VMEM((MROWS * D,), jnp.float32),
    ],
)
def _phase_m(tin_hbm, sidev_hbm, sidei_hbm, tout_hbm, svv, siv, tbuf):
    w = _wid()

    @pl.when(w < 16)
    def _():
        lo = w * MROWS

        def row(e):
            return [svv[pl.ds(e * D + 16 * k, 16)] for k in range(8)]

        def set_row(e, vals):
            for k in range(8):
                svv[pl.ds(e * D + 16 * k, 16)] = vals[k]

        pltpu.sync_copy(tin_hbm.at[pl.ds(lo * D, MROWS * D)], tbuf)
        pltpu.sync_copy(sidev_hbm, svv)
        pltpu.sync_copy(sidei_hbm, siv)
        sids = [siv[pl.ds(e * 16, 16)][0] for e in range(2 * NW)]
        for e in range(1, 2 * NW):
            same = sids[e] == sids[e - 1]
            pen = jnp.broadcast_to(
                jnp.where(same, jnp.float32(0.0), jnp.float32(-jnp.inf)), (16,))
            prev, cur = row(e - 1), row(e)
            set_row(e, [jnp.maximum(cur[k], prev[k] + pen) for k in range(8)])
        for e in range(2 * NW - 2, -1, -1):
            same = sids[e] == sids[e + 1]
            pen_c = jnp.broadcast_to(
                jnp.where(same, jnp.float32(-jnp.inf), jnp.float32(0.0)), (16,))
            pen_n = jnp.broadcast_to(
                jnp.where(same, jnp.float32(0.0), jnp.float32(-jnp.inf)), (16,))
            nxt, cur = row(e + 1), row(e)
            set_row(e, [jnp.maximum(cur[k] + pen_c, nxt[k] + pen_n)
                        for k in range(8)])
        for e in range(2 * NW):
            sid = sids[e]

            @pl.when((sid >= lo) & (sid < lo + MROWS))
            def _(e=e, sid=sid):
                off = (sid - lo) * D
                for k in range(8):
                    tbuf[pl.ds(off + 16 * k, 16)] = svv[pl.ds(e * D + 16 * k, 16)]

        pltpu.sync_copy(tbuf, tout_hbm.at[pl.ds(lo * D, MROWS * D)])


@functools.partial(
    pl.kernel,
    out_type=jax.ShapeDtypeStruct((N, 2, D), jnp.float32),
    mesh=_mesh,
    scratch_types=[
        pltpu.VMEM((CHUNK,), jnp.int32),
        pltpu.VMEM((2 * BLKB, D), jnp.float32),
        pltpu.VMEM((2 * BLKB, D), jnp.float32),
        pltpu.VMEM((2 * BLKB, 2, D), jnp.float32),
        pltpu.SemaphoreType.DMA,
        pltpu.SemaphoreType.DMA,
        pltpu.SemaphoreType.DMA,
    ],
)
def _phase_b(feat_hbm, ids_hbm, table_hbm, out_hbm,
             ids_v, fbuf, gbuf, obuf, sem_g, sem_f, sem_o):
    w = _wid()
    base = w * CHUNK
    pltpu.sync_copy(ids_hbm.at[pl.ds(base, CHUNK)], ids_v)

    pltpu.async_copy(
        table_hbm.at[ids_v.at[pl.ds(0, BLKB)]], gbuf.at[pl.ds(0, BLKB)], sem_g)
    pltpu.async_copy(
        feat_hbm.at[pl.ds(base, BLKB)], fbuf.at[pl.ds(0, BLKB)], sem_f)

    def blk_body(b, carry):
        slot = b % 2
        boff = slot * BLKB
        r0 = base + b * BLKB
        pltpu.make_async_copy(
            table_hbm.at[ids_v.at[pl.ds(0, BLKB)]],
            gbuf.at[pl.ds(boff, BLKB)], sem_g).wait()
        pltpu.make_async_copy(
            feat_hbm.at[pl.ds(base, BLKB)],
            fbuf.at[pl.ds(boff, BLKB)], sem_f).wait()

        @pl.when(b + 1 < NBLKB)
        def _():
            noff = (1 - slot) * BLKB
            pltpu.async_copy(
                table_hbm.at[ids_v.at[pl.ds((b + 1) * BLKB, BLKB)]],
                gbuf.at[pl.ds(noff, BLKB)], sem_g)
            pltpu.async_copy(
                feat_hbm.at[pl.ds(r0 + BLKB, BLKB)],
                fbuf.at[pl.ds(noff, BLKB)], sem_f)

        @pl.when(b >= 2)
        def _():
            pltpu.make_async_copy(
                obuf.at[pl.ds(boff, BLKB)],
                out_hbm.at[pl.ds(base, BLKB)], sem_o).wait()

        def row_body(j, carry):
            f = _vecs(fbuf, boff + j)
            g = _vecs(gbuf, boff + j)
            _store_row(obuf, [g[k] - f[k] for k in range(8)], boff + j, 0)
            _store_row(obuf, f, boff + j, 1)
            return carry

        lax.fori_loop(0, BLKB, row_body, carry)
        pltpu.async_copy(
            obuf.at[pl.ds(boff, BLKB)], out_hbm.at[pl.ds(r0, BLKB)], sem_o)
        return carry

    lax.fori_loop(0, NBLKB, blk_body, jnp.int32(0))
    for _ in range(2):
        pltpu.make_async_copy(
            obuf.at[pl.ds(0, BLKB)], out_hbm.at[pl.ds(base, BLKB)],
            sem_o).wait()


def kernel(feat, row_ids, num_rows):
    table, sidev, sidei = _phase_a(feat, row_ids)
    table = _phase_m(table, sidev, sidei)
    return _phase_b(feat, row_ids, table.reshape(R, D))


# speedup vs baseline: 2.0651x; 1.1082x over previous
import functools

import jax
import jax.numpy as jnp
from jax import lax
from jax.experimental import pallas as pl
from jax.experimental.pallas import tpu as pltpu
from jax.experimental.pallas import tpu_sc as plsc

N = 320000
D = 128
R = 10000
NC = 2
NS = 16
NW = NC * NS
CHUNK = N // NW
BLKA = 400
NBLKA = CHUNK // BLKA
BLKB = 80
NBLKB = CHUNK // BLKB
MROWS = R // 16

_mesh = plsc.VectorSubcoreMesh(
    core_axis_name="c", subcore_axis_name="s", num_cores=NC, num_subcores=NS)


def _wid():
    return lax.axis_index("s") * NC + lax.axis_index("c")


def _vecs(ref, *idx):
    return [ref[(*idx, pl.ds(16 * k, 16))] for k in range(8)]


def _store_row(ref, vals, *idx):
    for k in range(8):
        ref[(*idx, pl.ds(16 * k, 16))] = vals[k]


NSTAGE = 4
GRP = 16

@functools.partial(
    pl.kernel,
    out_type=(
        jax.ShapeDtypeStruct((R * D,), jnp.float32),
        jax.ShapeDtypeStruct((2 * NW * D,), jnp.float32),
        jax.ShapeDtypeStruct((2 * NW * 16,), jnp.int32),
    ),
    mesh=_mesh,
    scratch_types=[
        pltpu.VMEM((CHUNK + 16,), jnp.int32),
        pltpu.VMEM((2 * BLKA, D), jnp.float32),
        pltpu.VMEM((NSTAGE * D,), jnp.float32),
        pltpu.VMEM((D,), jnp.float32),
        pltpu.VMEM((D,), jnp.float32),
        pltpu.VMEM((32,), jnp.int32),
        pltpu.SMEM((8,), jnp.int32),
        pltpu.SemaphoreType.DMA,
        pltpu.SemaphoreType.DMA,
    ],
)
def _phase_a(feat_hbm, ids_hbm, table_hbm, sidev_hbm, sidei_hbm,
             ids_v, fbuf, stage, acc, first_acc, idstage, st, sem_f, sem_t):
    w = _wid()
    base = w * CHUNK
    pltpu.sync_copy(ids_hbm.at[pl.ds(base, CHUNK)], ids_v.at[pl.ds(0, CHUNK)])

    neg = jnp.full((16,), -jnp.inf, jnp.float32)
    _store_row(acc, [neg] * 8)
    st[0] = ids_v[pl.ds(0, 16)][0]
    st[1] = jnp.int32(0)

    def flush(accref):
        cur_id, nflush = st[0], st[1]

        @pl.when(nflush == 0)
        def _():
            for k in range(8):
                first_acc[pl.ds(16 * k, 16)] = accref[pl.ds(16 * k, 16)]

        @pl.when(nflush > 0)
        def _():
            @pl.when(nflush > NSTAGE)
            def _():
                pltpu.make_async_copy(
                    stage.at[pl.ds(0, D)], table_hbm.at[pl.ds(0, D)],
                    sem_t).wait()

            soff = ((nflush - 1) % NSTAGE) * D
            for k in range(8):
                stage[pl.ds(soff + 16 * k, 16)] = accref[pl.ds(16 * k, 16)]
            pltpu.async_copy(
                stage.at[pl.ds(soff, D)],
                table_hbm.at[pl.ds(cur_id * D, D)], sem_t)

        st[1] = nflush + 1

    pltpu.async_copy(
        feat_hbm.at[pl.ds(base, BLKA)], fbuf.at[pl.ds(0, BLKA)], sem_f)
    for blk in range(NBLKA):
        slot = blk % 2
        pltpu.make_async_copy(
            feat_hbm.at[pl.ds(base, BLKA)], fbuf.at[pl.ds(slot * BLKA, BLKA)],
            sem_f).wait()
        if blk + 1 < NBLKA:
            pltpu.async_copy(
                feat_hbm.at[pl.ds(base + (blk + 1) * BLKA, BLKA)],
                fbuf.at[pl.ds((1 - slot) * BLKA, BLKA)], sem_f)

        def grp_body(g, c, blk=blk, slot=slot):
            i = blk * BLKA + g * GRP
            fb = slot * BLKA + g * GRP
            idv = ids_v[pl.ds(i, 16)]
            last = idv[15]

            @pl.when(last == st[0])
            def _():
                for k in range(8):
                    m = acc[pl.ds(16 * k, 16)]
                    for j in range(GRP):
                        m = jnp.maximum(m, fbuf[fb + j, pl.ds(16 * k, 16)])
                    acc[pl.ds(16 * k, 16)] = m

            @pl.when(last != st[0])
            def _():
                def row_body(j, c2):
                    sid = ids_v[pl.ds(i + j, 16)][0]

                    @pl.when(sid != st[0])
                    def _():
                        flush(acc)
                        st[0] = sid
                        for k in range(8):
                            acc[pl.ds(16 * k, 16)] = \
                                fbuf[fb + j, pl.ds(16 * k, 16)]

                    @pl.when(sid == st[0])
                    def _():
                        for k in range(8):
                            acc[pl.ds(16 * k, 16)] = jnp.maximum(
                                acc[pl.ds(16 * k, 16)],
                                fbuf[fb + j, pl.ds(16 * k, 16)])

                    return c2

                lax.fori_loop(0, GRP, row_body, c)

            return c

        lax.fori_loop(0, BLKA // GRP, grp_body, jnp.int32(0))

    nflush = st[1]

    @pl.when(nflush == 0)
    def _():
        for k in range(8):
            first_acc[pl.ds(16 * k, 16)] = acc[pl.ds(16 * k, 16)]

    ndrain = jnp.minimum(jnp.maximum(nflush - 1, 0), NSTAGE)

    def drain(i, c):
        pltpu.make_async_copy(
            stage.at[pl.ds(0, D)], table_hbm.at[pl.ds(0, D)], sem_t).wait()
        return c

    lax.fori_loop(0, ndrain, drain, jnp.int32(0))

    pltpu.sync_copy(first_acc, sidev_hbm.at[pl.ds(2 * w * D, D)])
    pltpu.sync_copy(acc, sidev_hbm.at[pl.ds((2 * w + 1) * D, D)])
    idstage[pl.ds(0, 16)] = jnp.broadcast_to(ids_v[pl.ds(0, 16)][0], (16,))
    idstage[pl.ds(16, 16)] = jnp.broadcast_to(ids_v[pl.ds(CHUNK - 16, 16)][15], (16,))
    pltpu.sync_copy(idstage, sidei_hbm.at[pl.ds(2 * w * 16, 32)])


@functools.partial(
    pl.kernel,
    out_type=jax.ShapeDtypeStruct((R * D,), jnp.float32),
    mesh=_mesh,
    scratch_types=[
        pltpu.VMEM((2 * NW * D,), jnp.float32),
        pltpu.VMEM((2 * NW * 16,), jnp.int32),
        pltpu.VMEM((MROWS * D,), jnp.float32),
    ],
)
def _phase_m(tin_hbm, sidev_hbm, sidei_hbm, tout_hbm, svv, siv, tbuf):
    w = _wid()

    @pl.when(w < 16)
    def _():
        lo = w * MROWS

        def row(e):
            return [svv[pl.ds(e * D + 16 * k, 16)] for k in range(8)]

        def set_row(e, vals):
            for k in range(8):
                svv[pl.ds(e * D + 16 * k, 16)] = vals[k]

        pltpu.sync_copy(tin_hbm.at[pl.ds(lo * D, MROWS * D)], tbuf)
        pltpu.sync_copy(sidev_hbm, svv)
        pltpu.sync_copy(sidei_hbm, siv)
        sids = [siv[pl.ds(e * 16, 16)][0] for e in range(2 * NW)]
        for e in range(1, 2 * NW):
            same = sids[e] == sids[e - 1]
            pen = jnp.broadcast_to(
                jnp.where(same, jnp.float32(0.0), jnp.float32(-jnp.inf)), (16,))
            prev, cur = row(e - 1), row(e)
            set_row(e, [jnp.maximum(cur[k], prev[k] + pen) for k in range(8)])
        for e in range(2 * NW - 2, -1, -1):
            same = sids[e] == sids[e + 1]
            pen_c = jnp.broadcast_to(
                jnp.where(same, jnp.float32(-jnp.inf), jnp.float32(0.0)), (16,))
            pen_n = jnp.broadcast_to(
                jnp.where(same, jnp.float32(0.0), jnp.float32(-jnp.inf)), (16,))
            nxt, cur = row(e + 1), row(e)
            set_row(e, [jnp.maximum(cur[k] + pen_c, nxt[k] + pen_n)
                        for k in range(8)])
        for e in range(2 * NW):
            sid = sids[e]

            @pl.when((sid >= lo) & (sid < lo + MROWS))
            def _(e=e, sid=sid):
                off = (sid - lo) * D
                for k in range(8):
                    tbuf[pl.ds(off + 16 * k, 16)] = svv[pl.ds(e * D + 16 * k, 16)]

        pltpu.sync_copy(tbuf, tout_hbm.at[pl.ds(lo * D, MROWS * D)])


TROWS = 424

@functools.partial(
    pl.kernel,
    out_type=jax.ShapeDtypeStruct((N, 2 * D), jnp.float32),
    mesh=_mesh,
    scratch_types=[
        pltpu.VMEM((CHUNK + 16,), jnp.int32),
        pltpu.VMEM((3 * BLKB, 2 * D), jnp.float32),
        pltpu.VMEM((TROWS, D), jnp.float32),
        pltpu.SemaphoreType.DMA,
        pltpu.SemaphoreType.DMA,
        pltpu.SemaphoreType.DMA,
    ],
)
def _phase_b(feat_hbm, ids_hbm, table_hbm, out_hbm,
             ids_v, obuf, tbl, sem_f, sem_o, sem_g):
    w = _wid()
    base = w * CHUNK
    pltpu.sync_copy(ids_hbm.at[pl.ds(base, CHUNK)], ids_v.at[pl.ds(0, CHUNK)])
    first = ids_v[pl.ds(0, 16)][0]
    last = ids_v[pl.ds(CHUNK - 16, 16)][15]
    start = (jnp.minimum(first, R - TROWS) // 8) * 8
    rng_ok = (last - start) < TROWS

    def emit_row(ro, t):
        for k in range(8):
            obuf[ro, pl.ds(16 * k, 16)] = \
                t[k] - obuf[ro, pl.ds(D + 16 * k, 16)]

    @pl.when(rng_ok)
    def _():
        pltpu.async_copy(table_hbm.at[pl.ds(start, TROWS)], tbl, sem_g)
        pltpu.async_copy(
            feat_hbm.at[pl.ds(base, BLKB)],
            obuf.at[pl.ds(0, BLKB), pl.ds(D, D)], sem_f)
        pltpu.make_async_copy(
            table_hbm.at[pl.ds(start, TROWS)], tbl, sem_g).wait()

        def blk_body(b, c):
            slot = b % 3
            boff = slot * BLKB
            r0 = base + b * BLKB
            pltpu.make_async_copy(
                feat_hbm.at[pl.ds(base, BLKB)],
                obuf.at[pl.ds(0, BLKB), pl.ds(D, D)], sem_f).wait()

            @pl.when(b >= 2)
            def _():
                pltpu.make_async_copy(
                    obuf.at[pl.ds(0, BLKB)], out_hbm.at[pl.ds(base, BLKB)],
                    sem_o).wait()

            @pl.when(b + 1 < NBLKB)
            def _():
                noff = ((b + 1) % 3) * BLKB
                pltpu.async_copy(
                    feat_hbm.at[pl.ds(r0 + BLKB, BLKB)],
                    obuf.at[pl.ds(noff, BLKB), pl.ds(D, D)], sem_f)

            def grp_body(g, c2):
                i = b * BLKB + g * 16
                ro = boff + g * 16
                idv = ids_v[pl.ds(i, 16)]
                gf, gl = idv[0], idv[15]

                @pl.when(gf == gl)
                def _():
                    lid = gf - start
                    t = [tbl[lid, pl.ds(16 * k, 16)] for k in range(8)]
                    for j in range(16):
                        emit_row(ro + j, t)

                @pl.when(gf != gl)
                def _():
                    def row_body(j, c3):
                        sid = ids_v[pl.ds(i + j, 16)][0]
                        lid = sid - start
                        emit_row(ro + j,
                                 [tbl[lid, pl.ds(16 * k, 16)]
                                  for k in range(8)])
                        return c3

                    lax.fori_loop(0, 16, row_body, c2)

                return c2

            lax.fori_loop(0, BLKB // 16, grp_body, c)
            pltpu.async_copy(
                obuf.at[pl.ds(boff, BLKB)], out_hbm.at[pl.ds(r0, BLKB)], sem_o)
            return c

        lax.fori_loop(0, NBLKB, blk_body, jnp.int32(0))
        for _ in range(2):
            pltpu.make_async_copy(
                obuf.at[pl.ds(0, BLKB)], out_hbm.at[pl.ds(base, BLKB)],
                sem_o).wait()

    @pl.when(jnp.logical_not(rng_ok))
    def _():
        def blk_body(b, c):
            r0 = base + b * BLKB
            pltpu.sync_copy(
                feat_hbm.at[pl.ds(r0, BLKB)],
                obuf.at[pl.ds(0, BLKB), pl.ds(D, D)])
            pltpu.async_copy(
                table_hbm.at[ids_v.at[pl.ds(b * BLKB, BLKB)]],
                tbl.at[pl.ds(0, BLKB)], sem_g).wait()

            def row_body(j, c2):
                emit_row(j, [tbl[j, pl.ds(16 * k, 16)] for k in range(8)])
                return c2

            lax.fori_loop(0, BLKB, row_body, c)
            pltpu.sync_copy(
                obuf.at[pl.ds(0, BLKB)], out_hbm.at[pl.ds(r0, BLKB)])
            return c

        lax.fori_loop(0, NBLKB, blk_body, jnp.int32(0))


def kernel(feat, row_ids, num_rows):
    table, sidev, sidei = _phase_a(feat, row_ids)
    table = _phase_m(table, sidev, sidei)
    return _phase_b(feat, row_ids, table.reshape(R, D)).reshape(N, 2, D)
